# baseline (speedup 1.0000x reference)
import functools
import jax
import jax.numpy as jnp
from jax import lax
from jax.experimental import pallas as pl
from jax.experimental.pallas import tpu as pltpu
from jax.experimental.pallas import tpu_sc as plsc

_H = 512
_W = 512
_R = 9
_OUT = _H - _R + 1
_NT = 8
_BIG = 1 << 30
_SCALE = 1.0 / 81.0
_NEG = -3.0e38


def _pool_rows(win, nrows):
    vs = win[0:nrows, :]
    for k in range(1, _R):
        vs = vs + win[k:k + nrows, :]
    agg = vs[:, 0:_OUT]
    for k in range(1, _R):
        agg = agg + vs[:, k:k + _OUT]
    return agg * _SCALE


def _tc_init_body(x_ref, rmax_ref, rcode_ref):
    x = x_ref[0, 0]
    agg = _pool_rows(x, _OUT)
    rmax = jnp.max(agg, axis=1, keepdims=True)
    cio = lax.broadcasted_iota(jnp.int32, (_OUT, _OUT), 1)
    rcol = jnp.min(jnp.where(agg == rmax, cio, _BIG), axis=1, keepdims=True)
    rowio = lax.broadcasted_iota(jnp.int32, (_OUT, 1), 0)
    rcode = rowio * 1024 + rcol
    rmax_b = jnp.concatenate(
        [jnp.broadcast_to(rmax, (_OUT, 128)),
         jnp.full((_H - _OUT, 128), _NEG, jnp.float32)], axis=0)
    rcode_b = jnp.concatenate(
        [jnp.broadcast_to(rcode, (_OUT, 128)),
         jnp.full((_H - _OUT, 128), _BIG, jnp.int32)], axis=0)
    rmax_ref[0, 0, :] = rmax_b.T[0, :]
    rcode_ref[0, 0, :] = rcode_b.T[0, :]


def _tc_init(hm):
    B = hm.shape[0]
    return pl.pallas_call(
        _tc_init_body,
        grid=(B,),
        in_specs=[pl.BlockSpec((1, 1, _H, _W), lambda b: (b, 0, 0, 0))],
        out_specs=[
            pl.BlockSpec((1, 1, _H), lambda b: (b, 0, 0)),
            pl.BlockSpec((1, 1, _H), lambda b: (b, 0, 0)),
        ],
        out_shape=[
            jax.ShapeDtypeStruct((B, 1, _H), jnp.float32),
            jax.ShapeDtypeStruct((B, 1, _H), jnp.int32),
        ],
    )(hm)


def _sc_body(hm, rmaxh, rcodeh, out_c, work, win_v, vs_v, agg_v, rmax_v,
             rcode_v, out_v):
    cidx = lax.axis_index("c")
    sidx = lax.axis_index("s")
    wid = sidx * 2 + cidx
    io16 = lax.broadcasted_iota(jnp.int32, (16,), 0)

    @pl.when(wid < 8)
    def _():
        b = wid
        pltpu.sync_copy(rmaxh.at[b, 0], rmax_v)
        pltpu.sync_copy(rcodeh.at[b, 0], rcode_v)

        def cp(i, _):
            pltpu.sync_copy(hm.at[b, pl.ds(i * 32, 32)], win_v)
            pltpu.sync_copy(win_v, work.at[b, pl.ds(i * 32, 32)])
            return 0
        lax.fori_loop(0, 16, cp, 0)

        out_v[...] = jnp.zeros((16,), jnp.int32)
        vs_v[pl.ds(512, 16)] = jnp.zeros((16,), jnp.float32)

        def iter_body(t, _):
            def mx(j, acc):
                return jnp.maximum(acc, rmax_v[pl.ds(j * 16, 16)])
            acc = lax.fori_loop(0, 32, mx, jnp.full((16,), _NEG, jnp.float32))
            m = jnp.max(acc)

            def cd(j, acc):
                v = rmax_v[pl.ds(j * 16, 16)]
                cdv = rcode_v[pl.ds(j * 16, 16)]
                return jnp.minimum(acc, jnp.where(v == m, cdv, _BIG))
            cacc = lax.fori_loop(0, 32, cd, jnp.full((16,), _BIG, jnp.int32))
            code = jnp.min(cacc)
            r0 = code // 1024
            c0 = code - r0 * 1024

            ov = out_v[...]
            ov = jnp.where(io16 == 2 * t, c0 + 4, ov)
            ov = jnp.where(io16 == 2 * t + 1, r0 + 4, ov)
            out_v[...] = ov

            s2 = (jnp.minimum(jnp.maximum(r0 - 8, 0), 480) // 8) * 8
            pltpu.sync_copy(work.at[b, pl.ds(s2, 32)], win_v)

            cst = (c0 // 16) * 16
            cst2 = jnp.minimum(cst + 16, _W - 16)

            def zrow(i, _):
                j = r0 - s2 + i
                for st in (cst, cst2):
                    v = win_v[j, pl.ds(st, 16)]
                    gcol = st + io16
                    v = jnp.where((gcol >= c0) & (gcol < c0 + _R), 0.0, v)
                    win_v[j, pl.ds(st, 16)] = v
                return 0
            lax.fori_loop(0, _R, zrow, 0)
            pltpu.sync_copy(win_v, work.at[b, pl.ds(s2, 32)])

            def prow(jj, _):
                for kc in range(32):
                    a = win_v[jj, pl.ds(kc * 16, 16)]
                    for k in range(1, _R):
                        a = a + win_v[jj + k, pl.ds(kc * 16, 16)]
                    vs_v[pl.ds(kc * 16, 16)] = a
                for kc in range(32):
                    base = kc * 16
                    a = vs_v[pl.ds(base, 16)]
                    for k in range(1, _R):
                        a = a + vs_v[pl.ds(base + k, 16)]
                    agg_v[pl.ds(base, 16)] = a * _SCALE

                macc = jnp.full((16,), _NEG, jnp.float32)
                for kc in range(32):
                    v = agg_v[pl.ds(kc * 16, 16)]
                    if kc == 31:
                        v = jnp.where(io16 < 8, v, _NEG)
                    macc = jnp.maximum(macc, v)
                rm = jnp.max(macc)
                cacc2 = jnp.full((16,), _BIG, jnp.int32)
                for kc in range(32):
                    v = agg_v[pl.ds(kc * 16, 16)]
                    gcol = kc * 16 + io16
                    cacc2 = jnp.minimum(
                        cacc2, jnp.where((v == rm) & (gcol < _OUT), gcol, _BIG))
                rc = jnp.min(cacc2)

                g = s2 + jj
                kcd = (g // 16) * 16
                off = g - kcd
                mv = rmax_v[pl.ds(kcd, 16)]
                rmax_v[pl.ds(kcd, 16)] = jnp.where(io16 == off, rm, mv)
                cv = rcode_v[pl.ds(kcd, 16)]
                rcode_v[pl.ds(kcd, 16)] = jnp.where(io16 == off, g * 1024 + rc, cv)
                return 0
            lax.fori_loop(0, 24, prow, 0)
            return 0
        lax.fori_loop(0, _NT, iter_body, 0)
        pltpu.sync_copy(out_v, out_c.at[b])


def _sc_pick(hm3, rmaxh, rcodeh):
    mesh = plsc.VectorSubcoreMesh(core_axis_name="c", subcore_axis_name="s")
    f = functools.partial(
        pl.kernel,
        mesh=mesh,
        out_type=[
            jax.ShapeDtypeStruct((8, 16), jnp.int32),
            jax.ShapeDtypeStruct((8, _H, _W), jnp.float32),
        ],
        scratch_types=[
            pltpu.VMEM((32, _W), jnp.float32),
            pltpu.VMEM((528,), jnp.float32),
            pltpu.VMEM((_H,), jnp.float32),
            pltpu.VMEM((_H,), jnp.float32),
            pltpu.VMEM((_H,), jnp.int32),
            pltpu.VMEM((16,), jnp.int32),
        ],
        compiler_params=pltpu.CompilerParams(needs_layout_passes=False),
    )(_sc_body)
    return f(hm3, rmaxh, rcodeh)


def kernel(heatmap):
    B = heatmap.shape[0]
    hm3 = heatmap.reshape(B, _H, _W)
    rmaxh, rcodeh = _tc_init(heatmap)
    coords, _ = _sc_pick(hm3, rmaxh, rcodeh)
    return coords.reshape(B, _NT, 2)


# speedup vs baseline: 2.2549x; 2.2549x over previous
import functools
import jax
import jax.numpy as jnp
from jax import lax
from jax.experimental import pallas as pl
from jax.experimental.pallas import tpu as pltpu
from jax.experimental.pallas import tpu_sc as plsc

_H = 512
_W = 512
_R = 9
_OUT = _H - _R + 1
_NT = 8
_BIG = 1 << 30
_SCALE = 1.0 / 81.0
_NEG = -3.0e38


def _pool_rows(win, nrows):
    vs = win[0:nrows, :]
    for k in range(1, _R):
        vs = vs + win[k:k + nrows, :]
    agg = vs[:, 0:_OUT]
    for k in range(1, _R):
        agg = agg + vs[:, k:k + _OUT]
    return agg * _SCALE


def _tc_init_body(x_ref, rmax_ref, rcode_ref, segmax_ref, segarg_ref):
    x = x_ref[0, 0]
    aggT = _pool_rows(x.T, _OUT)
    aggTp = jnp.concatenate(
        [aggT, jnp.full((_H - _OUT, _OUT), _NEG, jnp.float32)], axis=0)
    colio = lax.broadcasted_iota(jnp.int32, (_H, _OUT), 0)
    sms, scs = [], []
    for s in range(32):
        blk = aggTp[16 * s:16 * s + 16, :]
        sm = jnp.max(blk, axis=0, keepdims=True)
        cio = colio[16 * s:16 * s + 16, :]
        sc_ = jnp.min(jnp.where(blk == sm, cio, _BIG), axis=0, keepdims=True)
        sms.append(sm)
        scs.append(sc_)
    segmaxT = jnp.concatenate(sms, axis=0)
    segargT = jnp.concatenate(scs, axis=0)
    lane_pad_f = jnp.full((32, _H - _OUT), _NEG, jnp.float32)
    lane_pad_i = jnp.full((32, _H - _OUT), _BIG, jnp.int32)
    segmax_ref[0] = jnp.concatenate([segmaxT, lane_pad_f], axis=1)
    segarg_ref[0] = jnp.concatenate([segargT, lane_pad_i], axis=1)

    rmax_row = jnp.max(segmaxT, axis=0, keepdims=True)
    colmin = jnp.min(jnp.where(segmaxT == rmax_row, segargT, _BIG),
                     axis=0, keepdims=True)
    rowio = lax.broadcasted_iota(jnp.int32, (1, _OUT), 1)
    rcode_row = rowio * 1024 + colmin
    rmax_ref[0, 0, :] = jnp.concatenate(
        [rmax_row, jnp.full((1, _H - _OUT), _NEG, jnp.float32)], axis=1)[0]
    rcode_ref[0, 0, :] = jnp.concatenate(
        [rcode_row, jnp.full((1, _H - _OUT), _BIG, jnp.int32)], axis=1)[0]


def _tc_init(hm):
    B = hm.shape[0]
    return pl.pallas_call(
        _tc_init_body,
        grid=(B,),
        in_specs=[pl.BlockSpec((1, 1, _H, _W), lambda b: (b, 0, 0, 0))],
        out_specs=[
            pl.BlockSpec((1, 1, _H), lambda b: (b, 0, 0)),
            pl.BlockSpec((1, 1, _H), lambda b: (b, 0, 0)),
            pl.BlockSpec((1, 32, _H), lambda b: (b, 0, 0)),
            pl.BlockSpec((1, 32, _H), lambda b: (b, 0, 0)),
        ],
        out_shape=[
            jax.ShapeDtypeStruct((B, 1, _H), jnp.float32),
            jax.ShapeDtypeStruct((B, 1, _H), jnp.int32),
            jax.ShapeDtypeStruct((B, 32, _H), jnp.float32),
            jax.ShapeDtypeStruct((B, 32, _H), jnp.int32),
        ],
    )(hm)


def _sc_body(hm, rmaxh, rcodeh, segmaxh, segargh, out_c, win_v, vs_v,
             segmax_v, segarg_v, rmax_v, rcode_v, blkmax_v, hist_v, out_v):
    cidx = lax.axis_index("c")
    sidx_ax = lax.axis_index("s")
    wid = sidx_ax * 2 + cidx
    io16 = lax.broadcasted_iota(jnp.int32, (16,), 0)

    @pl.when(wid < 8)
    def _():
        b = wid
        pltpu.sync_copy(rmaxh.at[b, 0], rmax_v)
        pltpu.sync_copy(rcodeh.at[b, 0], rcode_v)
        pltpu.sync_copy(segmaxh.at[b], segmax_v)
        pltpu.sync_copy(segargh.at[b], segarg_v)

        out_v[...] = jnp.zeros((16,), jnp.int32)
        hist_v[...] = jnp.full((16,), 1 << 21, jnp.int32)
        vs_v[pl.ds(64, 16)] = jnp.zeros((16,), jnp.float32)

        for half in range(2):
            acc = jnp.full((16,), _NEG, jnp.float32)
            for kb in range(16):
                blk = 16 * half + kb
                mkb = jnp.max(rmax_v[pl.ds(blk * 16, 16)])
                acc = jnp.where(io16 == kb, mkb, acc)
            blkmax_v[pl.ds(16 * half, 16)] = acc

        def iter_body(t, _):
            b0 = blkmax_v[pl.ds(0, 16)]
            b1 = blkmax_v[pl.ds(16, 16)]
            m = jnp.max(jnp.maximum(b0, b1))
            bi = jnp.min(jnp.minimum(jnp.where(b0 == m, io16, _BIG),
                                     jnp.where(b1 == m, io16 + 16, _BIG)))
            chunk = rmax_v[pl.ds(bi * 16, 16)]
            rloc = jnp.min(jnp.where(chunk == m, io16, _BIG))
            r0 = bi * 16 + rloc
            codes = rcode_v[pl.ds(bi * 16, 16)]
            code = jnp.min(jnp.where(io16 == rloc, codes, _BIG))
            c0 = code - r0 * 1024

            ov = out_v[...]
            ov = jnp.where(io16 == 2 * t, c0 + 4, ov)
            ov = jnp.where(io16 == 2 * t + 1, r0 + 4, ov)
            out_v[...] = ov
            hv = hist_v[...]
            hist_v[...] = jnp.where(io16 == t, code, hv)

            s2 = (jnp.minimum(jnp.maximum(r0 - 8, 0), 480) // 8) * 8
            cb = (jnp.minimum(jnp.maximum(c0 - 8, 0), 448) // 16) * 16
            pltpu.sync_copy(hm.at[b, pl.ds(s2, 32), pl.ds(cb, 64)], win_v)

            def replay(tt, _):
                hvv = hist_v[...]
                cp = jnp.min(jnp.where(io16 == tt, hvv, _BIG))
                rp = cp // 1024
                ccp = cp - rp * 1024
                inter = ((rp + _R > s2) & (rp < s2 + 32)
                         & (ccp + _R > cb) & (ccp < cb + 64))

                @pl.when(inter)
                def _():
                    for i in range(_R):
                        j = rp - s2 + i
                        ok = (j >= 0) & (j < 32)
                        jc = jnp.minimum(jnp.maximum(j, 0), 31)
                        for h in range(4):
                            v = win_v[jc, pl.ds(16 * h, 16)]
                            gcol = cb + 16 * h + io16
                            msk = ok & (gcol >= ccp) & (gcol < ccp + _R)
                            win_v[jc, pl.ds(16 * h, 16)] = jnp.where(msk, 0.0, v)
                return 0
            lax.fori_loop(0, t + 1, replay, 0)

            def prow(jj, _):
                g = s2 + jj
                gch = (g // 16) * 16
                goff = g - gch
                for h in range(4):
                    a = win_v[jj, pl.ds(16 * h, 16)]
                    for k in range(1, _R):
                        a = a + win_v[jj + k, pl.ds(16 * h, 16)]
                    vs_v[pl.ds(16 * h, 16)] = a
                for h in range(4):
                    base = 16 * h
                    a = vs_v[pl.ds(base, 16)]
                    for k in range(1, _R):
                        a = a + vs_v[pl.ds(base + k, 16)]
                    a = a * _SCALE
                    gcol = cb + base + io16
                    av = jnp.where(gcol < _OUT, a, _NEG)
                    sm = jnp.max(av)
                    scol = jnp.min(jnp.where(av == sm, gcol, _BIG))
                    sidx = cb // 16 + h
                    mvv = segmax_v[sidx, pl.ds(gch, 16)]
                    segmax_v[sidx, pl.ds(gch, 16)] = jnp.where(
                        io16 == goff, sm, mvv)
                    avv = segarg_v[sidx, pl.ds(gch, 16)]
                    segarg_v[sidx, pl.ds(gch, 16)] = jnp.where(
                        io16 == goff, scol, avv)

                gvec = jnp.full((16,), 0, jnp.int32) + g
                sv0 = plsc.load_gather(segmax_v, [io16, gvec])
                sv1 = plsc.load_gather(segmax_v, [io16 + 16, gvec])
                sa0 = plsc.load_gather(segarg_v, [io16, gvec])
                sa1 = plsc.load_gather(segarg_v, [io16 + 16, gvec])
                rm = jnp.max(jnp.maximum(sv0, sv1))
                colc = jnp.min(jnp.minimum(jnp.where(sv0 == rm, sa0, _BIG),
                                           jnp.where(sv1 == rm, sa1, _BIG)))
                mv = rmax_v[pl.ds(gch, 16)]
                rmax_v[pl.ds(gch, 16)] = jnp.where(io16 == goff, rm, mv)
                cv = rcode_v[pl.ds(gch, 16)]
                rcode_v[pl.ds(gch, 16)] = jnp.where(
                    io16 == goff, g * 1024 + colc, cv)
                return 0
            lax.fori_loop(0, 24, prow, 0)

            for u in range(2):
                bb = s2 // 16 + u
                mb = jnp.max(rmax_v[pl.ds(bb * 16, 16)])
                bch = (bb // 16) * 16
                bv = blkmax_v[pl.ds(bch, 16)]
                blkmax_v[pl.ds(bch, 16)] = jnp.where(
                    io16 == bb - bch, mb, bv)
            return 0
        lax.fori_loop(0, _NT, iter_body, 0)
        pltpu.sync_copy(out_v, out_c.at[b])


def _sc_pick(hm3, rmaxh, rcodeh, segmaxh, segargh):
    mesh = plsc.VectorSubcoreMesh(core_axis_name="c", subcore_axis_name="s")
    f = functools.partial(
        pl.kernel,
        mesh=mesh,
        out_type=jax.ShapeDtypeStruct((8, 16), jnp.int32),
        scratch_types=[
            pltpu.VMEM((32, 64), jnp.float32),
            pltpu.VMEM((80,), jnp.float32),
            pltpu.VMEM((32, _H), jnp.float32),
            pltpu.VMEM((32, _H), jnp.int32),
            pltpu.VMEM((_H,), jnp.float32),
            pltpu.VMEM((_H,), jnp.int32),
            pltpu.VMEM((32,), jnp.float32),
            pltpu.VMEM((16,), jnp.int32),
            pltpu.VMEM((16,), jnp.int32),
        ],
        compiler_params=pltpu.CompilerParams(needs_layout_passes=False,
                                             use_tc_tiling_on_sc=False),
    )(_sc_body)
    return f(hm3, rmaxh, rcodeh, segmaxh, segargh)


def kernel(heatmap):
    B = heatmap.shape[0]
    hm3 = heatmap.reshape(B, _H, _W)
    rmaxh, rcodeh, segmaxh, segargh = _tc_init(heatmap)
    coords = _sc_pick(hm3, rmaxh, rcodeh, segmaxh, segargh)
    return coords.reshape(B, _NT, 2)


# speedup vs baseline: 2.9241x; 1.2968x over previous
import functools
import jax
import jax.numpy as jnp
from jax import lax
from jax.experimental import pallas as pl
from jax.experimental.pallas import tpu as pltpu
from jax.experimental.pallas import tpu_sc as plsc

_H = 512
_W = 512
_R = 9
_OUT = _H - _R + 1
_NT = 8
_BIG = 1 << 30
_SCALE = 1.0 / 81.0
_NEG = -3.0e38


def _pool_rows(win, nrows):
    vs = win[0:nrows, :]
    for k in range(1, _R):
        vs = vs + win[k:k + nrows, :]
    agg = vs[:, 0:_OUT]
    for k in range(1, _R):
        agg = agg + vs[:, k:k + _OUT]
    return agg * _SCALE


def _tc_init_body(x_ref, rmax_ref, rcode_ref, segmax_ref, segarg_ref):
    x = x_ref[0, 0]
    aggT = _pool_rows(x.T, _OUT)
    aggTp = jnp.concatenate(
        [aggT, jnp.full((_H - _OUT, _OUT), _NEG, jnp.float32)], axis=0)
    colio = lax.broadcasted_iota(jnp.int32, (_H, _OUT), 0)
    sms, scs = [], []
    for s in range(32):
        blk = aggTp[16 * s:16 * s + 16, :]
        sm = jnp.max(blk, axis=0, keepdims=True)
        cio = colio[16 * s:16 * s + 16, :]
        sc_ = jnp.min(jnp.where(blk == sm, cio, _BIG), axis=0, keepdims=True)
        sms.append(sm)
        scs.append(sc_)
    segmaxT = jnp.concatenate(sms, axis=0)
    segargT = jnp.concatenate(scs, axis=0)
    lane_pad_f = jnp.full((32, _H - _OUT), _NEG, jnp.float32)
    lane_pad_i = jnp.full((32, _H - _OUT), _BIG, jnp.int32)
    segmax_ref[0] = jnp.concatenate([segmaxT, lane_pad_f], axis=1)
    segarg_ref[0] = jnp.concatenate([segargT, lane_pad_i], axis=1)

    rmax_row = jnp.max(segmaxT, axis=0, keepdims=True)
    colmin = jnp.min(jnp.where(segmaxT == rmax_row, segargT, _BIG),
                     axis=0, keepdims=True)
    rowio = lax.broadcasted_iota(jnp.int32, (1, _OUT), 1)
    rcode_row = rowio * 1024 + colmin
    rmax_ref[0, 0, :] = jnp.concatenate(
        [rmax_row, jnp.full((1, _H - _OUT), _NEG, jnp.float32)], axis=1)[0]
    rcode_ref[0, 0, :] = jnp.concatenate(
        [rcode_row, jnp.full((1, _H - _OUT), _BIG, jnp.int32)], axis=1)[0]


def _tc_init(hm):
    B = hm.shape[0]
    return pl.pallas_call(
        _tc_init_body,
        grid=(B,),
        in_specs=[pl.BlockSpec((1, 1, _H, _W), lambda b: (b, 0, 0, 0))],
        out_specs=[
            pl.BlockSpec((1, 1, _H), lambda b: (b, 0, 0)),
            pl.BlockSpec((1, 1, _H), lambda b: (b, 0, 0)),
            pl.BlockSpec((1, 32, _H), lambda b: (b, 0, 0)),
            pl.BlockSpec((1, 32, _H), lambda b: (b, 0, 0)),
        ],
        out_shape=[
            jax.ShapeDtypeStruct((B, 1, _H), jnp.float32),
            jax.ShapeDtypeStruct((B, 1, _H), jnp.int32),
            jax.ShapeDtypeStruct((B, 32, _H), jnp.float32),
            jax.ShapeDtypeStruct((B, 32, _H), jnp.int32),
        ],
    )(hm)


def _sc_body(hm, rmaxh, rcodeh, segmaxh, segargh, out_c, win_v, vs2_v,
             segmax_v, segarg_v, rmax_v, rcode_v, blkmax_v, hist_v, out_v):
    cidx = lax.axis_index("c")
    sidx_ax = lax.axis_index("s")
    wid = sidx_ax * 2 + cidx
    io16 = lax.broadcasted_iota(jnp.int32, (16,), 0)

    @pl.when(wid < 8)
    def _():
        b = wid
        pltpu.sync_copy(rmaxh.at[b, 0], rmax_v)
        pltpu.sync_copy(rcodeh.at[b, 0], rcode_v)
        pltpu.sync_copy(segmaxh.at[b], segmax_v)
        pltpu.sync_copy(segargh.at[b], segarg_v)

        out_v[...] = jnp.zeros((16,), jnp.int32)
        hist_v[...] = jnp.full((16,), 1 << 21, jnp.int32)

        for half in range(2):
            acc = jnp.full((16,), _NEG, jnp.float32)
            for kb in range(16):
                blk = 16 * half + kb
                mkb = jnp.max(rmax_v[pl.ds(blk * 16, 16)])
                acc = jnp.where(io16 == kb, mkb, acc)
            blkmax_v[pl.ds(16 * half, 16)] = acc

        def iter_body(t, _):
            b0 = blkmax_v[pl.ds(0, 16)]
            b1 = blkmax_v[pl.ds(16, 16)]
            m = jnp.max(jnp.maximum(b0, b1))
            bi = jnp.min(jnp.minimum(jnp.where(b0 == m, io16, _BIG),
                                     jnp.where(b1 == m, io16 + 16, _BIG)))
            chunk = rmax_v[pl.ds(bi * 16, 16)]
            rloc = jnp.min(jnp.where(chunk == m, io16, _BIG))
            r0 = bi * 16 + rloc
            codes = rcode_v[pl.ds(bi * 16, 16)]
            code = jnp.min(jnp.where(io16 == rloc, codes, _BIG))
            c0 = code - r0 * 1024

            ov = out_v[...]
            ov = jnp.where(io16 == 2 * t, c0 + 4, ov)
            ov = jnp.where(io16 == 2 * t + 1, r0 + 4, ov)
            out_v[...] = ov
            hv = hist_v[...]
            hist_v[...] = jnp.where(io16 == t, code, hv)

            rs = jnp.minimum(jnp.maximum(r0 - 8, 0), 487)
            s2 = (rs // 8) * 8
            jj0 = rs - s2
            sF = jnp.minimum(jnp.maximum(c0 - 8, 0), 488) // 16
            cb = 16 * sF
            pltpu.sync_copy(hm.at[b, pl.ds(s2, 32)], win_v)

            def replay(tt, _):
                hvv = hist_v[...]
                cp = jnp.min(jnp.where(io16 == tt, hvv, _BIG))
                rp = cp // 1024
                ccp = cp - rp * 1024
                cpb = (ccp // 16) * 16
                cpb2 = jnp.minimum(cpb + 16, _W - 16)
                inter = (rp + _R > s2) & (rp < s2 + 32)

                @pl.when(inter)
                def _():
                    for i in range(_R):
                        j = rp - s2 + i
                        ok = (j >= 0) & (j < 32)
                        jc = jnp.minimum(jnp.maximum(j, 0), 31)
                        for st in (cpb, cpb2):
                            v = win_v[jc, pl.ds(st, 16)]
                            gcol = st + io16
                            msk = ok & (gcol >= ccp) & (gcol < ccp + _R)
                            win_v[jc, pl.ds(st, 16)] = jnp.where(msk, 0.0, v)
                return 0
            lax.fori_loop(0, t + 1, replay, 0)

            for h in range(3):
                cbh = jnp.minimum(cb + 16 * h, _W - 16)
                vrows = [win_v[jj0 + i, pl.ds(cbh, 16)] for i in range(25)]
                for j in range(17):
                    a = vrows[j]
                    for k in range(1, _R):
                        a = a + vrows[j + k]
                    vs2_v[j, pl.ds(16 * h, 16)] = a

            def prow(jj, _):
                g = rs + jj
                gch = (g // 16) * 16
                goff = g - gch
                for h in range(2):
                    base = 16 * h
                    a = vs2_v[jj, pl.ds(base, 16)]
                    for k in range(1, _R):
                        a = a + vs2_v[jj, pl.ds(base + k, 16)]
                    a = a * _SCALE
                    gcol = cb + base + io16
                    av = jnp.where(gcol < _OUT, a, _NEG)
                    sm = jnp.max(av)
                    scol = jnp.min(jnp.where(av == sm, gcol, _BIG))
                    sidx = sF + h
                    mvv = segmax_v[sidx, pl.ds(gch, 16)]
                    segmax_v[sidx, pl.ds(gch, 16)] = jnp.where(
                        io16 == goff, sm, mvv)
                    avv = segarg_v[sidx, pl.ds(gch, 16)]
                    segarg_v[sidx, pl.ds(gch, 16)] = jnp.where(
                        io16 == goff, scol, avv)

                gvec = jnp.full((16,), 0, jnp.int32) + g
                sv0 = plsc.load_gather(segmax_v, [io16, gvec])
                sv1 = plsc.load_gather(segmax_v, [io16 + 16, gvec])
                sa0 = plsc.load_gather(segarg_v, [io16, gvec])
                sa1 = plsc.load_gather(segarg_v, [io16 + 16, gvec])
                rm = jnp.max(jnp.maximum(sv0, sv1))
                colc = jnp.min(jnp.minimum(jnp.where(sv0 == rm, sa0, _BIG),
                                           jnp.where(sv1 == rm, sa1, _BIG)))
                mv = rmax_v[pl.ds(gch, 16)]
                rmax_v[pl.ds(gch, 16)] = jnp.where(io16 == goff, rm, mv)
                cv = rcode_v[pl.ds(gch, 16)]
                rcode_v[pl.ds(gch, 16)] = jnp.where(
                    io16 == goff, g * 1024 + colc, cv)
                return 0
            lax.fori_loop(0, 17, prow, 0)

            for u in range(3):
                bb = jnp.minimum(rs // 16 + u, 31)
                mb = jnp.max(rmax_v[pl.ds(bb * 16, 16)])
                bch = (bb // 16) * 16
                bv = blkmax_v[pl.ds(bch, 16)]
                blkmax_v[pl.ds(bch, 16)] = jnp.where(
                    io16 == bb - bch, mb, bv)
            return 0
        lax.fori_loop(0, _NT, iter_body, 0)
        pltpu.sync_copy(out_v, out_c.at[b])


def _sc_pick(hm3, rmaxh, rcodeh, segmaxh, segargh):
    mesh = plsc.VectorSubcoreMesh(core_axis_name="c", subcore_axis_name="s")
    f = functools.partial(
        pl.kernel,
        mesh=mesh,
        out_type=jax.ShapeDtypeStruct((8, 16), jnp.int32),
        scratch_types=[
            pltpu.VMEM((32, _W), jnp.float32),
            pltpu.VMEM((17, 64), jnp.float32),
            pltpu.VMEM((32, _H), jnp.float32),
            pltpu.VMEM((32, _H), jnp.int32),
            pltpu.VMEM((_H,), jnp.float32),
            pltpu.VMEM((_H,), jnp.int32),
            pltpu.VMEM((32,), jnp.float32),
            pltpu.VMEM((16,), jnp.int32),
            pltpu.VMEM((16,), jnp.int32),
        ],
        compiler_params=pltpu.CompilerParams(needs_layout_passes=False),
    )(_sc_body)
    return f(hm3, rmaxh, rcodeh, segmaxh, segargh)


def kernel(heatmap):
    B = heatmap.shape[0]
    hm3 = heatmap.reshape(B, _H, _W)
    rmaxh, rcodeh, segmaxh, segargh = _tc_init(heatmap)
    coords = _sc_pick(hm3, rmaxh, rcodeh, segmaxh, segargh)
    return coords.reshape(B, _NT, 2)


# speedup vs baseline: 3.0051x; 1.0277x over previous
import functools
import jax
import jax.numpy as jnp
from jax import lax
from jax.experimental import pallas as pl
from jax.experimental.pallas import tpu as pltpu
from jax.experimental.pallas import tpu_sc as plsc

_H = 512
_W = 512
_R = 9
_OUT = _H - _R + 1
_NT = 8
_BIG = 1 << 30
_SCALE = 1.0 / 81.0
_NEG = -3.0e38


def _pool_rows(win, nrows):
    vs = win[0:nrows, :]
    for k in range(1, _R):
        vs = vs + win[k:k + nrows, :]
    agg = vs[:, 0:_OUT]
    for k in range(1, _R):
        agg = agg + vs[:, k:k + _OUT]
    return agg * _SCALE


def _tc_init_body(x_ref, rmax_ref, rcode_ref, segmax_ref, segarg_ref):
    x = x_ref[0, 0]
    aggT = _pool_rows(x.T, _OUT)
    aggTp = jnp.concatenate(
        [aggT, jnp.full((_H - _OUT, _OUT), _NEG, jnp.float32)], axis=0)
    colio = lax.broadcasted_iota(jnp.int32, (_H, _OUT), 0)
    sms, scs = [], []
    for s in range(32):
        blk = aggTp[16 * s:16 * s + 16, :]
        sm = jnp.max(blk, axis=0, keepdims=True)
        cio = colio[16 * s:16 * s + 16, :]
        sc_ = jnp.min(jnp.where(blk == sm, cio, _BIG), axis=0, keepdims=True)
        sms.append(sm)
        scs.append(sc_)
    segmaxT = jnp.concatenate(sms, axis=0)
    segargT = jnp.concatenate(scs, axis=0)
    lane_pad_f = jnp.full((32, _H - _OUT), _NEG, jnp.float32)
    lane_pad_i = jnp.full((32, _H - _OUT), _BIG, jnp.int32)
    segmax_ref[0] = jnp.concatenate([segmaxT, lane_pad_f], axis=1)
    segarg_ref[0] = jnp.concatenate([segargT, lane_pad_i], axis=1)

    rmax_row = jnp.max(segmaxT, axis=0, keepdims=True)
    colmin = jnp.min(jnp.where(segmaxT == rmax_row, segargT, _BIG),
                     axis=0, keepdims=True)
    rowio = lax.broadcasted_iota(jnp.int32, (1, _OUT), 1)
    rcode_row = rowio * 1024 + colmin
    rmax_ref[0, 0, :] = jnp.concatenate(
        [rmax_row, jnp.full((1, _H - _OUT), _NEG, jnp.float32)], axis=1)[0]
    rcode_ref[0, 0, :] = jnp.concatenate(
        [rcode_row, jnp.full((1, _H - _OUT), _BIG, jnp.int32)], axis=1)[0]


def _tc_init(hm):
    B = hm.shape[0]
    return pl.pallas_call(
        _tc_init_body,
        grid=(B,),
        in_specs=[pl.BlockSpec((1, 1, _H, _W), lambda b: (b, 0, 0, 0))],
        out_specs=[
            pl.BlockSpec((1, 1, _H), lambda b: (b, 0, 0)),
            pl.BlockSpec((1, 1, _H), lambda b: (b, 0, 0)),
            pl.BlockSpec((1, 32, _H), lambda b: (b, 0, 0)),
            pl.BlockSpec((1, 32, _H), lambda b: (b, 0, 0)),
        ],
        out_shape=[
            jax.ShapeDtypeStruct((B, 1, _H), jnp.float32),
            jax.ShapeDtypeStruct((B, 1, _H), jnp.int32),
            jax.ShapeDtypeStruct((B, 32, _H), jnp.float32),
            jax.ShapeDtypeStruct((B, 32, _H), jnp.int32),
        ],
    )(hm)


def _sc_body(hm, rmaxh, rcodeh, segmaxh, segargh, out_c, win_v, vs2_v,
             segmax_v, segarg_v, rmax_v, rcode_v, blkmax_v, hist_v, out_v):
    cidx = lax.axis_index("c")
    sidx_ax = lax.axis_index("s")
    wid = sidx_ax * 2 + cidx
    io16 = lax.broadcasted_iota(jnp.int32, (16,), 0)

    @pl.when(wid < 8)
    def _():
        b = wid
        pltpu.sync_copy(rmaxh.at[b, 0], rmax_v)
        pltpu.sync_copy(rcodeh.at[b, 0], rcode_v)
        pltpu.sync_copy(segmaxh.at[b], segmax_v)
        pltpu.sync_copy(segargh.at[b], segarg_v)

        out_v[...] = jnp.zeros((16,), jnp.int32)
        hist_v[...] = jnp.full((16,), 1 << 21, jnp.int32)

        for half in range(2):
            acc = jnp.full((16,), _NEG, jnp.float32)
            for kb in range(16):
                blk = 16 * half + kb
                mkb = jnp.max(rmax_v[pl.ds(blk * 16, 16)])
                acc = jnp.where(io16 == kb, mkb, acc)
            blkmax_v[pl.ds(16 * half, 16)] = acc

        def iter_body(t, _):
            b0 = blkmax_v[pl.ds(0, 16)]
            b1 = blkmax_v[pl.ds(16, 16)]
            m = jnp.max(jnp.maximum(b0, b1))
            bi = jnp.min(jnp.minimum(jnp.where(b0 == m, io16, _BIG),
                                     jnp.where(b1 == m, io16 + 16, _BIG)))
            chunk = rmax_v[pl.ds(bi * 16, 16)]
            rloc = jnp.min(jnp.where(chunk == m, io16, _BIG))
            r0 = bi * 16 + rloc
            codes = rcode_v[pl.ds(bi * 16, 16)]
            code = jnp.min(jnp.where(io16 == rloc, codes, _BIG))
            c0 = code - r0 * 1024

            ov = out_v[...]
            ov = jnp.where(io16 == 2 * t, c0 + 4, ov)
            ov = jnp.where(io16 == 2 * t + 1, r0 + 4, ov)
            out_v[...] = ov
            hv = hist_v[...]
            hist_v[...] = jnp.where(io16 == t, code, hv)

            rs = jnp.minimum(jnp.maximum(r0 - 8, 0), 487)
            s2 = (rs // 8) * 8
            jj0 = rs - s2
            sF = jnp.minimum(jnp.maximum(c0 - 8, 0), 488) // 16
            cb = 16 * sF
            cw = jnp.minimum(jnp.maximum((c0 - 8) // 128, 0), 2) * 128
            pltpu.sync_copy(hm.at[b, pl.ds(s2, 32), pl.ds(cw, 256)], win_v)

            def replay(tt, _):
                hvv = hist_v[...]
                cp = jnp.min(jnp.where(io16 == tt, hvv, _BIG))
                rp = cp // 1024
                ccp = cp - rp * 1024
                cpb = (ccp // 16) * 16
                st1 = jnp.minimum(jnp.maximum(cpb - cw, 0), 240)
                st2 = jnp.minimum(jnp.maximum(cpb + 16 - cw, 0), 240)
                inter = (rp + _R > s2) & (rp < s2 + 32)

                @pl.when(inter)
                def _():
                    for i in range(_R):
                        j = rp - s2 + i
                        ok = (j >= 0) & (j < 32)
                        jc = jnp.minimum(jnp.maximum(j, 0), 31)
                        for st in (st1, st2):
                            v = win_v[jc, pl.ds(st, 16)]
                            gcol = cw + st + io16
                            msk = ok & (gcol >= ccp) & (gcol < ccp + _R)
                            win_v[jc, pl.ds(st, 16)] = jnp.where(msk, 0.0, v)
                return 0
            lax.fori_loop(0, t + 1, replay, 0)

            for h in range(3):
                cbh = jnp.minimum(cb + 16 * h, _W - 16) - cw
                vrows = [win_v[jj0 + i, pl.ds(cbh, 16)] for i in range(25)]
                for j in range(17):
                    a = vrows[j]
                    for k in range(1, _R):
                        a = a + vrows[j + k]
                    vs2_v[j, pl.ds(16 * h, 16)] = a

            def row_work(jj):
                g = rs + jj
                gch = (g // 16) * 16
                goff = g - gch
                for h in range(2):
                    base = 16 * h
                    a = vs2_v[jj, pl.ds(base, 16)]
                    for k in range(1, _R):
                        a = a + vs2_v[jj, pl.ds(base + k, 16)]
                    a = a * _SCALE
                    gcol = cb + base + io16
                    av = jnp.where(gcol < _OUT, a, _NEG)
                    sm = jnp.max(av)
                    scol = jnp.min(jnp.where(av == sm, gcol, _BIG))
                    sidx = sF + h
                    mvv = segmax_v[sidx, pl.ds(gch, 16)]
                    segmax_v[sidx, pl.ds(gch, 16)] = jnp.where(
                        io16 == goff, sm, mvv)
                    avv = segarg_v[sidx, pl.ds(gch, 16)]
                    segarg_v[sidx, pl.ds(gch, 16)] = jnp.where(
                        io16 == goff, scol, avv)

                gvec = jnp.full((16,), 0, jnp.int32) + g
                sv0 = plsc.load_gather(segmax_v, [io16, gvec])
                sv1 = plsc.load_gather(segmax_v, [io16 + 16, gvec])
                sa0 = plsc.load_gather(segarg_v, [io16, gvec])
                sa1 = plsc.load_gather(segarg_v, [io16 + 16, gvec])
                rm = jnp.max(jnp.maximum(sv0, sv1))
                colc = jnp.min(jnp.minimum(jnp.where(sv0 == rm, sa0, _BIG),
                                           jnp.where(sv1 == rm, sa1, _BIG)))
                mv = rmax_v[pl.ds(gch, 16)]
                rmax_v[pl.ds(gch, 16)] = jnp.where(io16 == goff, rm, mv)
                cv = rcode_v[pl.ds(gch, 16)]
                rcode_v[pl.ds(gch, 16)] = jnp.where(
                    io16 == goff, g * 1024 + colc, cv)

            def prow2(ii, _):
                row_work(2 * ii)
                row_work(2 * ii + 1)
                return 0
            lax.fori_loop(0, 8, prow2, 0)
            row_work(16)

            for u in range(3):
                bb = jnp.minimum(rs // 16 + u, 31)
                mb = jnp.max(rmax_v[pl.ds(bb * 16, 16)])
                bch = (bb // 16) * 16
                bv = blkmax_v[pl.ds(bch, 16)]
                blkmax_v[pl.ds(bch, 16)] = jnp.where(
                    io16 == bb - bch, mb, bv)
            return 0
        lax.fori_loop(0, _NT, iter_body, 0)
        pltpu.sync_copy(out_v, out_c.at[b])


def _sc_pick(hm3, rmaxh, rcodeh, segmaxh, segargh):
    mesh = plsc.VectorSubcoreMesh(core_axis_name="c", subcore_axis_name="s")
    f = functools.partial(
        pl.kernel,
        mesh=mesh,
        out_type=jax.ShapeDtypeStruct((8, 16), jnp.int32),
        scratch_types=[
            pltpu.VMEM((32, 256), jnp.float32),
            pltpu.VMEM((17, 64), jnp.float32),
            pltpu.VMEM((32, _H), jnp.float32),
            pltpu.VMEM((32, _H), jnp.int32),
            pltpu.VMEM((_H,), jnp.float32),
            pltpu.VMEM((_H,), jnp.int32),
            pltpu.VMEM((32,), jnp.float32),
            pltpu.VMEM((16,), jnp.int32),
            pltpu.VMEM((16,), jnp.int32),
        ],
        compiler_params=pltpu.CompilerParams(needs_layout_passes=False),
    )(_sc_body)
    return f(hm3, rmaxh, rcodeh, segmaxh, segargh)


def kernel(heatmap):
    B = heatmap.shape[0]
    hm3 = heatmap.reshape(B, _H, _W)
    rmaxh, rcodeh, segmaxh, segargh = _tc_init(heatmap)
    coords = _sc_pick(hm3, rmaxh, rcodeh, segmaxh, segargh)
    return coords.reshape(B, _NT, 2)


# speedup vs baseline: 3.0730x; 1.0226x over previous
import functools
import jax
import jax.numpy as jnp
from jax import lax
from jax.experimental import pallas as pl
from jax.experimental.pallas import tpu as pltpu
from jax.experimental.pallas import tpu_sc as plsc

_H = 512
_W = 512
_R = 9
_OUT = _H - _R + 1
_NT = 8
_BIG = 1 << 30
_SCALE = 1.0 / 81.0
_NEG = -3.0e38


def _pool_rows(win, nrows):
    vs = win[0:nrows, :]
    for k in range(1, _R):
        vs = vs + win[k:k + nrows, :]
    agg = vs[:, 0:_OUT]
    for k in range(1, _R):
        agg = agg + vs[:, k:k + _OUT]
    return agg * _SCALE


def _tc_init_body(x_ref, rmax_ref, rcode_ref, segmax_ref, segarg_ref):
    x = x_ref[0, 0]
    aggT = _pool_rows(x.T, _OUT)
    aggTp = jnp.concatenate(
        [aggT, jnp.full((_H - _OUT, _OUT), _NEG, jnp.float32)], axis=0)
    colio = lax.broadcasted_iota(jnp.int32, (_H, _OUT), 0)
    sms, scs = [], []
    for s in range(32):
        blk = aggTp[16 * s:16 * s + 16, :]
        sm = jnp.max(blk, axis=0, keepdims=True)
        cio = colio[16 * s:16 * s + 16, :]
        sc_ = jnp.min(jnp.where(blk == sm, cio, _BIG), axis=0, keepdims=True)
        sms.append(sm)
        scs.append(sc_)
    segmaxT = jnp.concatenate(sms, axis=0)
    segargT = jnp.concatenate(scs, axis=0)
    lane_pad_f = jnp.full((32, _H - _OUT), _NEG, jnp.float32)
    lane_pad_i = jnp.full((32, _H - _OUT), _BIG, jnp.int32)
    segmax_ref[0] = jnp.concatenate([segmaxT, lane_pad_f], axis=1)
    segarg_ref[0] = jnp.concatenate([segargT, lane_pad_i], axis=1)

    rmax_row = jnp.max(segmaxT, axis=0, keepdims=True)
    colmin = jnp.min(jnp.where(segmaxT == rmax_row, segargT, _BIG),
                     axis=0, keepdims=True)
    rowio = lax.broadcasted_iota(jnp.int32, (1, _OUT), 1)
    rcode_row = rowio * 1024 + colmin
    rmax_ref[0, 0, :] = jnp.concatenate(
        [rmax_row, jnp.full((1, _H - _OUT), _NEG, jnp.float32)], axis=1)[0]
    rcode_ref[0, 0, :] = jnp.concatenate(
        [rcode_row, jnp.full((1, _H - _OUT), _BIG, jnp.int32)], axis=1)[0]


def _tc_init(hm):
    B = hm.shape[0]
    return pl.pallas_call(
        _tc_init_body,
        grid=(B,),
        in_specs=[pl.BlockSpec((1, 1, _H, _W), lambda b: (b, 0, 0, 0))],
        out_specs=[
            pl.BlockSpec((1, 1, _H), lambda b: (b, 0, 0)),
            pl.BlockSpec((1, 1, _H), lambda b: (b, 0, 0)),
            pl.BlockSpec((1, 32, _H), lambda b: (b, 0, 0)),
            pl.BlockSpec((1, 32, _H), lambda b: (b, 0, 0)),
        ],
        out_shape=[
            jax.ShapeDtypeStruct((B, 1, _H), jnp.float32),
            jax.ShapeDtypeStruct((B, 1, _H), jnp.int32),
            jax.ShapeDtypeStruct((B, 32, _H), jnp.float32),
            jax.ShapeDtypeStruct((B, 32, _H), jnp.int32),
        ],
    )(hm)


def _sc_body(hm, rmaxh, rcodeh, segmaxh, segargh, out_c, win_v, vs2_v,
             segmax_v, segarg_v, rmax_v, rcode_v, blkmax_v, hist_v, out_v,
             dsem):
    cidx = lax.axis_index("c")
    sidx_ax = lax.axis_index("s")
    wid = sidx_ax * 2 + cidx
    io16 = lax.broadcasted_iota(jnp.int32, (16,), 0)

    @pl.when(wid < 8)
    def _():
        b = wid
        cp1 = pltpu.async_copy(rmaxh.at[b, 0], rmax_v, dsem)
        cp2 = pltpu.async_copy(rcodeh.at[b, 0], rcode_v, dsem)
        cp3 = pltpu.async_copy(segmaxh.at[b], segmax_v, dsem)
        cp4 = pltpu.async_copy(segargh.at[b], segarg_v, dsem)
        cp1.wait()
        cp2.wait()
        cp3.wait()
        cp4.wait()

        out_v[...] = jnp.zeros((16,), jnp.int32)
        hist_v[...] = jnp.full((16,), 1 << 21, jnp.int32)

        for half in range(2):
            acc = jnp.full((16,), _NEG, jnp.float32)
            for kb in range(16):
                blk = 16 * half + kb
                mkb = jnp.max(rmax_v[pl.ds(blk * 16, 16)])
                acc = jnp.where(io16 == kb, mkb, acc)
            blkmax_v[pl.ds(16 * half, 16)] = acc

        def iter_body(t, _):
            b0 = blkmax_v[pl.ds(0, 16)]
            b1 = blkmax_v[pl.ds(16, 16)]
            m = jnp.max(jnp.maximum(b0, b1))
            bi = jnp.min(jnp.minimum(jnp.where(b0 == m, io16, _BIG),
                                     jnp.where(b1 == m, io16 + 16, _BIG)))
            chunk = rmax_v[pl.ds(bi * 16, 16)]
            rloc = jnp.min(jnp.where(chunk == m, io16, _BIG))
            r0 = bi * 16 + rloc
            codes = rcode_v[pl.ds(bi * 16, 16)]
            code = jnp.min(jnp.where(io16 == rloc, codes, _BIG))
            c0 = code - r0 * 1024

            ov = out_v[...]
            ov = jnp.where(io16 == 2 * t, c0 + 4, ov)
            ov = jnp.where(io16 == 2 * t + 1, r0 + 4, ov)
            out_v[...] = ov
            hv = hist_v[...]
            hist_v[...] = jnp.where(io16 == t, code, hv)

            rs = jnp.minimum(jnp.maximum(r0 - 8, 0), 487)
            s2 = (rs // 8) * 8
            jj0 = rs - s2
            sF = jnp.minimum(jnp.maximum(c0 - 8, 0), 488) // 16
            cb = 16 * sF
            cw = jnp.minimum(jnp.maximum((c0 - 8) // 128, 0), 2) * 128
            pltpu.sync_copy(hm.at[b, pl.ds(s2, 32), pl.ds(cw, 256)], win_v)

            def replay(tt, _):
                hvv = hist_v[...]
                cp = jnp.min(jnp.where(io16 == tt, hvv, _BIG))
                rp = cp // 1024
                ccp = cp - rp * 1024
                cpb = (ccp // 16) * 16
                st1 = jnp.minimum(jnp.maximum(cpb - cw, 0), 240)
                st2 = jnp.minimum(jnp.maximum(cpb + 16 - cw, 0), 240)
                inter = (rp + _R > s2) & (rp < s2 + 32)

                @pl.when(inter)
                def _():
                    for i in range(_R):
                        j = rp - s2 + i
                        ok = (j >= 0) & (j < 32)
                        jc = jnp.minimum(jnp.maximum(j, 0), 31)
                        for st in (st1, st2):
                            v = win_v[jc, pl.ds(st, 16)]
                            gcol = cw + st + io16
                            msk = ok & (gcol >= ccp) & (gcol < ccp + _R)
                            win_v[jc, pl.ds(st, 16)] = jnp.where(msk, 0.0, v)
                return 0
            lax.fori_loop(0, t + 1, replay, 0)

            for h in range(3):
                cbh = jnp.minimum(cb + 16 * h, _W - 16) - cw
                vrows = [win_v[jj0 + i, pl.ds(cbh, 16)] for i in range(25)]
                for j in range(17):
                    a = vrows[j]
                    for k in range(1, _R):
                        a = a + vrows[j + k]
                    vs2_v[j, pl.ds(16 * h, 16)] = a

            def row_work(jj):
                g = rs + jj
                gch = (g // 16) * 16
                goff = g - gch
                for h in range(2):
                    base = 16 * h
                    a = vs2_v[jj, pl.ds(base, 16)]
                    for k in range(1, _R):
                        a = a + vs2_v[jj, pl.ds(base + k, 16)]
                    a = a * _SCALE
                    gcol = cb + base + io16
                    av = jnp.where(gcol < _OUT, a, _NEG)
                    sm = jnp.max(av)
                    scol = jnp.min(jnp.where(av == sm, gcol, _BIG))
                    sidx = sF + h
                    mvv = segmax_v[sidx, pl.ds(gch, 16)]
                    segmax_v[sidx, pl.ds(gch, 16)] = jnp.where(
                        io16 == goff, sm, mvv)
                    avv = segarg_v[sidx, pl.ds(gch, 16)]
                    segarg_v[sidx, pl.ds(gch, 16)] = jnp.where(
                        io16 == goff, scol, avv)

                gvec = jnp.full((16,), 0, jnp.int32) + g
                sv0 = plsc.load_gather(segmax_v, [io16, gvec])
                sv1 = plsc.load_gather(segmax_v, [io16 + 16, gvec])
                sa0 = plsc.load_gather(segarg_v, [io16, gvec])
                sa1 = plsc.load_gather(segarg_v, [io16 + 16, gvec])
                rm = jnp.max(jnp.maximum(sv0, sv1))
                colc = jnp.min(jnp.minimum(jnp.where(sv0 == rm, sa0, _BIG),
                                           jnp.where(sv1 == rm, sa1, _BIG)))
                mv = rmax_v[pl.ds(gch, 16)]
                rmax_v[pl.ds(gch, 16)] = jnp.where(io16 == goff, rm, mv)
                cv = rcode_v[pl.ds(gch, 16)]
                rcode_v[pl.ds(gch, 16)] = jnp.where(
                    io16 == goff, g * 1024 + colc, cv)

            def prow2(ii, _):
                row_work(2 * ii)
                row_work(2 * ii + 1)
                return 0
            lax.fori_loop(0, 8, prow2, 0)
            row_work(16)

            for u in range(3):
                bb = jnp.minimum(rs // 16 + u, 31)
                mb = jnp.max(rmax_v[pl.ds(bb * 16, 16)])
                bch = (bb // 16) * 16
                bv = blkmax_v[pl.ds(bch, 16)]
                blkmax_v[pl.ds(bch, 16)] = jnp.where(
                    io16 == bb - bch, mb, bv)
            return 0
        lax.fori_loop(0, _NT, iter_body, 0)
        pltpu.sync_copy(out_v, out_c.at[b])


def _sc_pick(hm3, rmaxh, rcodeh, segmaxh, segargh):
    mesh = plsc.VectorSubcoreMesh(core_axis_name="c", subcore_axis_name="s")
    f = functools.partial(
        pl.kernel,
        mesh=mesh,
        out_type=jax.ShapeDtypeStruct((8, 16), jnp.int32),
        scratch_types=[
            pltpu.VMEM((32, 256), jnp.float32),
            pltpu.VMEM((17, 64), jnp.float32),
            pltpu.VMEM((32, _H), jnp.float32),
            pltpu.VMEM((32, _H), jnp.int32),
            pltpu.VMEM((_H,), jnp.float32),
            pltpu.VMEM((_H,), jnp.int32),
            pltpu.VMEM((32,), jnp.float32),
            pltpu.VMEM((16,), jnp.int32),
            pltpu.VMEM((16,), jnp.int32),
            pltpu.SemaphoreType.DMA,
        ],
        compiler_params=pltpu.CompilerParams(needs_layout_passes=False),
    )(_sc_body)
    return f(hm3, rmaxh, rcodeh, segmaxh, segargh)


def kernel(heatmap):
    B = heatmap.shape[0]
    hm3 = heatmap.reshape(B, _H, _W)
    rmaxh, rcodeh, segmaxh, segargh = _tc_init(heatmap)
    coords = _sc_pick(hm3, rmaxh, rcodeh, segmaxh, segargh)
    return coords.reshape(B, _NT, 2)
